# baseline (speedup 1.0000x reference)
import functools

import jax
import jax.numpy as jnp
from jax import lax
from jax.experimental import pallas as pl
from jax.experimental.pallas import tpu as pltpu
from jax.experimental.pallas import tpu_sc as plsc


def _sc_mask(w, m):
    n = w.size
    num_cores, num_subcores = 2, 16
    nw = num_cores * num_subcores
    per_w = n // nw
    ch = 32768
    n_ch = per_w // ch
    mesh = plsc.VectorSubcoreMesh(
        core_axis_name="c", subcore_axis_name="s",
        num_cores=num_cores, num_subcores=num_subcores)

    @functools.partial(
        pl.kernel,
        out_type=jax.ShapeDtypeStruct((n,), jnp.float32),
        mesh=mesh,
        scratch_types=[
            pltpu.VMEM((ch,), jnp.float32),
            pltpu.VMEM((ch,), jnp.float32),
        ],
    )
    def body(w_hbm, m_hbm, o_hbm, w_v, m_v):
        wid = lax.axis_index("s") * num_cores + lax.axis_index("c")
        base = wid * per_w

        def chunk(ci, carry):
            off = base + ci * ch
            pltpu.sync_copy(w_hbm.at[pl.ds(off, ch)], w_v)
            pltpu.sync_copy(m_hbm.at[pl.ds(off, ch)], m_v)

            def vec(i, c):
                j = i * 128
                for u in range(8):
                    jj = j + u * 16
                    w_v[pl.ds(jj, 16)] = (w_v[pl.ds(jj, 16)]
                                          * m_v[pl.ds(jj, 16)])
                return c

            lax.fori_loop(0, ch // 128, vec, 0)

            pltpu.sync_copy(w_v, o_hbm.at[pl.ds(off, ch)])
            return carry

        lax.fori_loop(0, n_ch, chunk, 0)

    return body(w.reshape(-1), m.reshape(-1)).reshape(w.shape)


def _mm_body(*refs, k_steps, out_dtype, has_mask, has_bias):
    it = iter(refs)
    a_ref = next(it)
    w_ref = next(it)
    m_ref = next(it) if has_mask else None
    b_ref = next(it) if has_bias else None
    o_ref = next(it)
    acc_ref = next(it) if k_steps > 1 else None
    k = pl.program_id(2)

    def _dot():
        a = a_ref[...].astype(jnp.bfloat16)
        s = w_ref[...]
        if has_mask:
            s = s * m_ref[...]
        s = s.astype(jnp.bfloat16)
        return jax.lax.dot_general(
            a, s, (((1,), (1,)), ((), ())), preferred_element_type=jnp.float32)

    def _finish(x):
        if has_bias:
            x = x + b_ref[...]
        return x.astype(out_dtype)

    if k_steps == 1:
        o_ref[...] = _finish(_dot())
        return

    @pl.when(k == 0)
    def _():
        acc_ref[...] = _dot()

    @pl.when((k > 0) & (k < k_steps - 1))
    def _():
        acc_ref[...] += _dot()

    @pl.when(k == k_steps - 1)
    def _():
        o_ref[...] = _finish(acc_ref[...] + _dot())


def _masked_mm(a, w, mask, bias, out_dtype, bm, bn, bk):
    m_dim, k_dim = a.shape
    n_dim = w.shape[0]
    grid = (m_dim // bm, n_dim // bn, k_dim // bk)
    in_specs = [
        pl.BlockSpec((bm, bk), lambda m, n, k: (m, k)),
        pl.BlockSpec((bn, bk), lambda m, n, k: (n, k)),
    ]
    args = [a, w]
    if mask is not None:
        in_specs.append(pl.BlockSpec((bn, bk), lambda m, n, k: (n, k)))
        args.append(mask)
    if bias is not None:
        in_specs.append(pl.BlockSpec((1, bn), lambda m, n, k: (0, n)))
        args.append(bias.reshape(1, -1))
    body = functools.partial(
        _mm_body, k_steps=grid[2], out_dtype=out_dtype,
        has_mask=mask is not None, has_bias=bias is not None)
    return pl.pallas_call(
        body,
        grid=grid,
        in_specs=in_specs,
        out_specs=pl.BlockSpec((bm, bn), lambda m, n, k: (m, n)),
        out_shape=jax.ShapeDtypeStruct((m_dim, n_dim), out_dtype),
        scratch_shapes=(
            [pltpu.VMEM((bm, bn), jnp.float32)] if grid[2] > 1 else []),
        compiler_params=pltpu.CompilerParams(
            dimension_semantics=("arbitrary", "arbitrary", "arbitrary")),
    )(*args)


def kernel(U, W0, W1, W2, M0, M1, M2, bias):
    s1 = _sc_mask(W1, M1)
    s0 = _sc_mask(W0, M0)
    a1 = _masked_mm(U, W2, M2, None, jnp.bfloat16, 2048, 1024, 1024)
    a2 = _masked_mm(a1, s1, None, None, jnp.bfloat16, 2048, 1024, 2048)
    out = _masked_mm(a2, s0, None, bias, jnp.float32, 2048, 1024, 1024)
    return out


# speedup vs baseline: 1.4952x; 1.4952x over previous
import functools

import jax
import jax.numpy as jnp
from jax import lax
from jax.experimental import pallas as pl
from jax.experimental.pallas import tpu as pltpu
from jax.experimental.pallas import tpu_sc as plsc


def _sc_mask(w, m):
    rows, cols = w.shape
    num_cores, num_subcores = 2, 16
    nw = num_cores * num_subcores
    rows_per_w = rows // nw
    ch = 16
    n_ch = rows_per_w // ch
    mesh = plsc.VectorSubcoreMesh(
        core_axis_name="c", subcore_axis_name="s",
        num_cores=num_cores, num_subcores=num_subcores)

    @functools.partial(
        pl.kernel,
        out_type=jax.ShapeDtypeStruct((rows, cols), jnp.float32),
        mesh=mesh,
        compiler_params=pltpu.CompilerParams(use_tc_tiling_on_sc=True),
        scratch_types=[
            pltpu.VMEM((ch, cols), jnp.float32),
            pltpu.VMEM((ch, cols), jnp.float32),
        ],
    )
    def body(w_hbm, m_hbm, o_hbm, w_v, m_v):
        wid = lax.axis_index("s") * num_cores + lax.axis_index("c")
        base = wid * rows_per_w

        def chunk(ci, carry):
            r0 = base + ci * ch
            pltpu.sync_copy(w_hbm.at[pl.ds(r0, ch)], w_v)
            pltpu.sync_copy(m_hbm.at[pl.ds(r0, ch)], m_v)

            def vec(i, c):
                j = i * 64
                for r in range(ch):
                    for u in range(4):
                        jj = j + u * 16
                        w_v[r, pl.ds(jj, 16)] = (w_v[r, pl.ds(jj, 16)]
                                                 * m_v[r, pl.ds(jj, 16)])
                return c

            lax.fori_loop(0, cols // 64, vec, 0)

            pltpu.sync_copy(w_v, o_hbm.at[pl.ds(r0, ch)])
            return carry

        lax.fori_loop(0, n_ch, chunk, 0)

    return body(w, m)


def _mm_body(*refs, k_steps, out_dtype, has_mask, has_bias):
    it = iter(refs)
    a_ref = next(it)
    w_ref = next(it)
    m_ref = next(it) if has_mask else None
    b_ref = next(it) if has_bias else None
    o_ref = next(it)
    acc_ref = next(it) if k_steps > 1 else None
    k = pl.program_id(2)

    def _dot():
        a = a_ref[...].astype(jnp.bfloat16)
        s = w_ref[...]
        if has_mask:
            s = s * m_ref[...]
        s = s.astype(jnp.bfloat16)
        return jax.lax.dot_general(
            a, s, (((1,), (1,)), ((), ())), preferred_element_type=jnp.float32)

    def _finish(x):
        if has_bias:
            x = x + b_ref[...]
        return x.astype(out_dtype)

    if k_steps == 1:
        o_ref[...] = _finish(_dot())
        return

    @pl.when(k == 0)
    def _():
        acc_ref[...] = _dot()

    @pl.when((k > 0) & (k < k_steps - 1))
    def _():
        acc_ref[...] += _dot()

    @pl.when(k == k_steps - 1)
    def _():
        o_ref[...] = _finish(acc_ref[...] + _dot())


def _masked_mm(a, w, mask, bias, out_dtype, bm, bn, bk):
    m_dim, k_dim = a.shape
    n_dim = w.shape[0]
    grid = (m_dim // bm, n_dim // bn, k_dim // bk)
    in_specs = [
        pl.BlockSpec((bm, bk), lambda m, n, k: (m, k)),
        pl.BlockSpec((bn, bk), lambda m, n, k: (n, k)),
    ]
    args = [a, w]
    if mask is not None:
        in_specs.append(pl.BlockSpec((bn, bk), lambda m, n, k: (n, k)))
        args.append(mask)
    if bias is not None:
        in_specs.append(pl.BlockSpec((1, bn), lambda m, n, k: (0, n)))
        args.append(bias.reshape(1, -1))
    body = functools.partial(
        _mm_body, k_steps=grid[2], out_dtype=out_dtype,
        has_mask=mask is not None, has_bias=bias is not None)
    return pl.pallas_call(
        body,
        grid=grid,
        in_specs=in_specs,
        out_specs=pl.BlockSpec((bm, bn), lambda m, n, k: (m, n)),
        out_shape=jax.ShapeDtypeStruct((m_dim, n_dim), out_dtype),
        scratch_shapes=(
            [pltpu.VMEM((bm, bn), jnp.float32)] if grid[2] > 1 else []),
        compiler_params=pltpu.CompilerParams(
            dimension_semantics=("arbitrary", "arbitrary", "arbitrary")),
    )(*args)


def kernel(U, W0, W1, W2, M0, M1, M2, bias):
    s1 = _sc_mask(W1, M1)
    s0 = _sc_mask(W0, M0)
    a1 = _masked_mm(U, W2, M2, None, jnp.bfloat16, 2048, 1024, 1024)
    a2 = _masked_mm(a1, s1, None, None, jnp.bfloat16, 2048, 1024, 2048)
    out = _masked_mm(a2, s0, None, bias, jnp.float32, 2048, 1024, 1024)
    return out


# speedup vs baseline: 1.5859x; 1.0607x over previous
import functools

import jax
import jax.numpy as jnp
from jax import lax
from jax.experimental import pallas as pl
from jax.experimental.pallas import tpu as pltpu
from jax.experimental.pallas import tpu_sc as plsc


def _sc_mask(w, m):
    rows, cols = w.shape
    num_cores, num_subcores = 2, 16
    nw = num_cores * num_subcores
    rows_per_w = rows // nw
    ch = 16
    n_ch = rows_per_w // ch
    mesh = plsc.VectorSubcoreMesh(
        core_axis_name="c", subcore_axis_name="s",
        num_cores=num_cores, num_subcores=num_subcores)

    @functools.partial(
        pl.kernel,
        out_type=jax.ShapeDtypeStruct((rows, cols), jnp.float32),
        mesh=mesh,
        compiler_params=pltpu.CompilerParams(use_tc_tiling_on_sc=True),
        scratch_types=[
            pltpu.VMEM((ch, cols), jnp.float32),
            pltpu.VMEM((ch, cols), jnp.float32),
        ],
    )
    def body(w_hbm, m_hbm, o_hbm, w_v, m_v):
        wid = lax.axis_index("s") * num_cores + lax.axis_index("c")
        base = wid * rows_per_w

        def chunk(ci, carry):
            r0 = base + ci * ch
            pltpu.sync_copy(w_hbm.at[pl.ds(r0, ch)], w_v)
            pltpu.sync_copy(m_hbm.at[pl.ds(r0, ch)], m_v)

            def vec(i, c):
                j = i * 64
                for r in range(ch):
                    for u in range(4):
                        jj = j + u * 16
                        w_v[r, pl.ds(jj, 16)] = (w_v[r, pl.ds(jj, 16)]
                                                 * m_v[r, pl.ds(jj, 16)])
                return c

            lax.fori_loop(0, cols // 64, vec, 0)

            pltpu.sync_copy(w_v, o_hbm.at[pl.ds(r0, ch)])
            return carry

        lax.fori_loop(0, n_ch, chunk, 0)

    return body(w, m)


def _mm_body(*refs, k_steps, out_dtype, has_mask, has_bias):
    it = iter(refs)
    a_ref = next(it)
    w_ref = next(it)
    m_ref = next(it) if has_mask else None
    b_ref = next(it) if has_bias else None
    o_ref = next(it)
    acc_ref = next(it) if k_steps > 1 else None
    k = pl.program_id(2)

    def _dot():
        a = a_ref[...].astype(jnp.bfloat16)
        s = w_ref[...]
        if has_mask:
            s = s * m_ref[...]
        s = s.astype(jnp.bfloat16)
        return jax.lax.dot_general(
            a, s, (((1,), (1,)), ((), ())), preferred_element_type=jnp.float32)

    def _finish(x):
        if has_bias:
            x = x + b_ref[...]
        return x.astype(out_dtype)

    if k_steps == 1:
        o_ref[...] = _finish(_dot())
        return

    @pl.when(k == 0)
    def _():
        acc_ref[...] = _dot()

    @pl.when((k > 0) & (k < k_steps - 1))
    def _():
        acc_ref[...] += _dot()

    @pl.when(k == k_steps - 1)
    def _():
        o_ref[...] = _finish(acc_ref[...] + _dot())


def _masked_mm(a, w, mask, bias, out_dtype, bm, bn, bk):
    m_dim, k_dim = a.shape
    n_dim = w.shape[0]
    grid = (m_dim // bm, n_dim // bn, k_dim // bk)
    in_specs = [
        pl.BlockSpec((bm, bk), lambda m, n, k: (m, k)),
        pl.BlockSpec((bn, bk), lambda m, n, k: (n, k)),
    ]
    args = [a, w]
    if mask is not None:
        in_specs.append(pl.BlockSpec((bn, bk), lambda m, n, k: (n, k)))
        args.append(mask)
    if bias is not None:
        in_specs.append(pl.BlockSpec((1, bn), lambda m, n, k: (0, n)))
        args.append(bias.reshape(1, -1))
    body = functools.partial(
        _mm_body, k_steps=grid[2], out_dtype=out_dtype,
        has_mask=mask is not None, has_bias=bias is not None)
    return pl.pallas_call(
        body,
        grid=grid,
        in_specs=in_specs,
        out_specs=pl.BlockSpec((bm, bn), lambda m, n, k: (m, n)),
        out_shape=jax.ShapeDtypeStruct((m_dim, n_dim), out_dtype),
        scratch_shapes=(
            [pltpu.VMEM((bm, bn), jnp.float32)] if grid[2] > 1 else []),
        compiler_params=pltpu.CompilerParams(
            dimension_semantics=("arbitrary", "arbitrary", "arbitrary")),
    )(*args)


def kernel(U, W0, W1, W2, M0, M1, M2, bias):
    s1 = _sc_mask(W1, M1)
    a1 = _masked_mm(U, W2, M2, None, jnp.bfloat16, 2048, 1024, 1024)
    a2 = _masked_mm(a1, s1, None, None, jnp.bfloat16, 2048, 1024, 2048)
    out = _masked_mm(a2, W0, M0, bias, jnp.float32, 2048, 1024, 1024)
    return out
